# baseline (speedup 1.0000x reference)
import functools

import jax
import jax.numpy as jnp
from jax import lax
from jax.experimental import pallas as pl
from jax.experimental.pallas import tpu as pltpu
from jax.experimental.pallas import tpu_sc as plsc

_N = 10000
_D = 128
_E = 320000

_NC = 2
_NS = 16
_NW = _NC * _NS
_NSLOT = 4
_CH = 96
_NCHUNK = 104
_EPW = _E // _NW
_TAIL = _EPW - _NCHUNK * _CH
_ROWS = 10112
_RPS = _ROWS // _NS
def _mk_blocks():
    b, r0 = [], 0
    while r0 < _RPS:
        n = min(_CH, _RPS - r0)
        b.append((r0, n))
        r0 += n
    return tuple(b)


_BLOCKS = _mk_blocks()


def _segsum_body(h_hbm, src_hbm, dst_hbm, out_hbm, acc, *scr):
    slots = tuple(scr[5 * k:5 * k + 5] for k in range(_NSLOT))
    rows0, g0 = slots[0][0], slots[0][3]
    c = lax.axis_index("c")
    s = lax.axis_index("s")
    wid = s * _NC + c

    def _zrow(r, _):
        def _zcol(k, __):
            rows0[r, pl.ds(k * 16, 16)] = jnp.zeros((16,), jnp.float32)
            return 0
        return lax.fori_loop(0, _D // 16, _zcol, 0)
    lax.fori_loop(0, _CH, _zrow, 0)
    for r0b, nb in _BLOCKS:
        pltpu.async_copy(rows0.at[pl.ds(0, nb)],
                         acc.at[pl.ds(s * _RPS + r0b, nb)], g0)
    for r0b, nb in _BLOCKS:
        pltpu.make_async_copy(rows0.at[pl.ds(0, nb)],
                              acc.at[pl.ds(s * _RPS + r0b, nb)], g0).wait()
    plsc.subcore_barrier()

    e0 = wid * _EPW
    for k, (rw, isk, idk, gk, sk) in enumerate(slots):
        off = e0 + k * _CH
        pltpu.sync_copy(src_hbm.at[pl.ds(off, _CH)], isk)
        pltpu.sync_copy(dst_hbm.at[pl.ds(off, _CH)], idk)
        pltpu.async_copy(h_hbm.at[isk], rw, gk)

    def _iter(i, _):
        for rw, isk, idk, gk, sk in slots:
            pltpu.make_async_copy(h_hbm.at[isk], rw, gk).wait()
            pltpu.async_copy(rw, acc.at[idk], sk, add=True)
        for k, (rw, isk, idk, gk, sk) in enumerate(slots):
            jn = _NSLOT * i + k + _NSLOT

            @pl.when(jn < _NCHUNK)
            def _(rw=rw, isk=isk, idk=idk, gk=gk, sk=sk, jn=jn):
                pltpu.make_async_copy(rw, acc.at[idk], sk).wait()
                off = e0 + jn * _CH
                pltpu.sync_copy(src_hbm.at[pl.ds(off, _CH)], isk)
                pltpu.sync_copy(dst_hbm.at[pl.ds(off, _CH)], idk)
                pltpu.async_copy(h_hbm.at[isk], rw, gk)
        return 0
    lax.fori_loop(0, _NCHUNK // _NSLOT, _iter, 0)
    for rw, isk, idk, gk, sk in slots:
        pltpu.make_async_copy(rw, acc.at[idk], sk).wait()

    is0, id0 = slots[0][1], slots[0][2]
    ot = e0 + _NCHUNK * _CH
    pltpu.sync_copy(src_hbm.at[pl.ds(ot, _TAIL)], is0.at[pl.ds(0, _TAIL)])
    pltpu.sync_copy(dst_hbm.at[pl.ds(ot, _TAIL)], id0.at[pl.ds(0, _TAIL)])
    iv = lax.iota(jnp.int32, 16)
    for k in range(_TAIL, _CH, 16):
        is0[pl.ds(k, 16)] = iv
        id0[pl.ds(k, 16)] = _N + (k - _TAIL) + iv
    pltpu.async_copy(h_hbm.at[is0], rows0, g0).wait()
    pltpu.sync_copy(rows0, acc.at[id0], add=True)
    plsc.subcore_barrier()

    def _wb_in(idx):
        r0b, nb = _BLOCKS[idx]
        rw = slots[idx % _NSLOT][0]
        return (acc.at[pl.ds(s * _RPS + r0b, nb)], rw.at[pl.ds(0, nb)],
                slots[idx % _NSLOT][3])
    def _wb_out(idx):
        r0b, nb = _BLOCKS[idx]
        rw = slots[idx % _NSLOT][0]
        return (rw.at[pl.ds(0, nb)], out_hbm.at[c, pl.ds(s * _RPS + r0b, nb)],
                slots[idx % _NSLOT][4])
    for idx in range(len(_BLOCKS)):
        if idx >= _NSLOT:
            pltpu.make_async_copy(*_wb_out(idx - _NSLOT)).wait()
        pltpu.async_copy(*_wb_in(idx))
        pltpu.make_async_copy(*_wb_in(idx)).wait()
        pltpu.async_copy(*_wb_out(idx))
    for idx in range(max(0, len(_BLOCKS) - _NSLOT), len(_BLOCKS)):
        pltpu.make_async_copy(*_wb_out(idx)).wait()


def _segsum(h, src, dst):
    mesh = plsc.VectorSubcoreMesh(core_axis_name="c", subcore_axis_name="s")
    f = pl.kernel(
        _segsum_body,
        mesh=mesh,
        out_type=jax.ShapeDtypeStruct((_NC, _ROWS, _D), jnp.float32),
        scratch_types=[
            pltpu.VMEM_SHARED((_ROWS, _D), jnp.float32),
        ] + [
            t for _k in range(_NSLOT) for t in (
                pltpu.VMEM((_CH, _D), jnp.float32),
                pltpu.VMEM((_CH,), jnp.int32),
                pltpu.VMEM((_CH,), jnp.int32),
                pltpu.SemaphoreType.DMA,
                pltpu.SemaphoreType.DMA,
            )
        ],
    )
    return f(h, src, dst)


def _matmul_t(x, w):
    return lax.dot_general(x, w, (((1,), (1,)), ((), ())),
                           preferred_element_type=jnp.float32)


def _lin_body(x_ref, w_ref, b_ref, o_ref):
    o_ref[...] = _matmul_t(x_ref[...], w_ref[...]) + b_ref[...]


def _lin(x, p):
    return pl.pallas_call(
        _lin_body,
        out_shape=jax.ShapeDtypeStruct((_N, _D), jnp.float32),
    )(x, p["W"], p["b"].reshape(1, -1))


def _mlp_core(x_ref, a_ref, eps_ref, w1_ref, b1_ref, g_ref, be_ref, w2_ref, b2_ref):
    agg = a_ref[0, 0:_N, :] + a_ref[1, 0:_N, :]
    h = (1.0 + eps_ref[0]) * x_ref[...] + agg
    t = _matmul_t(h, w1_ref[...]) + b1_ref[...]
    mean = jnp.mean(t, axis=0, keepdims=True)
    var = jnp.mean((t - mean) ** 2, axis=0, keepdims=True)
    t = (t - mean) * lax.rsqrt(var + 1e-5) * g_ref[...] + be_ref[...]
    t = jnp.maximum(t, 0.0)
    t = _matmul_t(t, w2_ref[...]) + b2_ref[...]
    return jnp.maximum(t, 0.0)


def _gin_mlp_body(x_ref, a_ref, eps_ref, w1_ref, b1_ref, g_ref, be_ref,
                  w2_ref, b2_ref, o_ref):
    o_ref[...] = _mlp_core(x_ref, a_ref, eps_ref, w1_ref, b1_ref, g_ref,
                           be_ref, w2_ref, b2_ref)


def _gin_mlp_final_body(x_ref, a_ref, eps_ref, w1_ref, b1_ref, g_ref, be_ref,
                        w2_ref, b2_ref, wf_ref, bf_ref, o_ref):
    t = _mlp_core(x_ref, a_ref, eps_ref, w1_ref, b1_ref, g_ref,
                  be_ref, w2_ref, b2_ref)
    o_ref[...] = _matmul_t(t, wf_ref[...]) + bf_ref[...]


def _mlp_args(x, agg, p):
    return (x, agg, p["eps"].reshape(1),
            p["W1"], p["b1"].reshape(1, -1),
            p["gamma"].reshape(1, -1), p["beta"].reshape(1, -1),
            p["W2"], p["b2"].reshape(1, -1))


_SMEM1 = pl.BlockSpec(memory_space=pltpu.SMEM)


def _gin_mlp(x, agg, p):
    specs = [None, None, _SMEM1] + [None] * 6
    specs = [s if s is not None else pl.BlockSpec() for s in specs]
    return pl.pallas_call(
        _gin_mlp_body,
        in_specs=specs,
        out_shape=jax.ShapeDtypeStruct((_N, _D), jnp.float32),
    )(*_mlp_args(x, agg, p))


def _gin_mlp_final(x, agg, p, pf):
    specs = [None, None, _SMEM1] + [None] * 8
    specs = [s if s is not None else pl.BlockSpec() for s in specs]
    return pl.pallas_call(
        _gin_mlp_final_body,
        in_specs=specs,
        out_shape=jax.ShapeDtypeStruct((_N, pf["W"].shape[0]), jnp.float32),
    )(*_mlp_args(x, agg, p), pf["W"], pf["b"].reshape(1, -1))


def kernel(x_author, x_paper, params, ei_writes, ei_written):
    p = params
    src_w, dst_w = ei_writes[0], ei_writes[1]
    src_n, dst_n = ei_written[0], ei_written[1]

    h_a = _lin(x_author, p["lin_author"])
    h_p = _lin(x_paper, p["lin_paper"])

    l1, l2 = p["layers"][0], p["layers"][1]
    agg_p = _segsum(h_a, src_w, dst_w)
    agg_a = _segsum(h_p, src_n, dst_n)
    h_p1 = _gin_mlp(h_p, agg_p, l1["writes"])
    h_a1 = _gin_mlp(h_a, agg_a, l1["written"])

    agg_a2 = _segsum(h_p1, src_n, dst_n)
    return _gin_mlp_final(h_a1, agg_a2, l2["written"], p["final"])


# speedup vs baseline: 1.0884x; 1.0884x over previous
import functools

import jax
import jax.numpy as jnp
from jax import lax
from jax.experimental import pallas as pl
from jax.experimental.pallas import tpu as pltpu
from jax.experimental.pallas import tpu_sc as plsc

_N = 10000
_D = 128
_E = 320000

_NC = 2
_NS = 16
_NW = _NC * _NS
_NSLOT = 3
_CH = 128
_NCHUNK = 78
_EPW = _E // _NW
_TAIL = _EPW - _NCHUNK * _CH
_ROWS = 10112
_RPS = _ROWS // _NS
def _mk_blocks():
    b, r0 = [], 0
    while r0 < _RPS:
        n = min(_CH, _RPS - r0)
        b.append((r0, n))
        r0 += n
    return tuple(b)


_BLOCKS = _mk_blocks()


def _segsum_body(h_hbm, src_hbm, dst_hbm, out_hbm, acc, *scr):
    slots = tuple(scr[5 * k:5 * k + 5] for k in range(_NSLOT))
    ist, idt = scr[5 * _NSLOT], scr[5 * _NSLOT + 1]
    rows0, g0 = slots[0][0], slots[0][3]
    c = lax.axis_index("c")
    s = lax.axis_index("s")
    wid = s * _NC + c

    def _zrow(r, _):
        def _zcol(k, __):
            rows0[r, pl.ds(k * 16, 16)] = jnp.zeros((16,), jnp.float32)
            return 0
        return lax.fori_loop(0, _D // 16, _zcol, 0)
    lax.fori_loop(0, _CH, _zrow, 0)
    for r0b, nb in _BLOCKS:
        pltpu.async_copy(rows0.at[pl.ds(0, nb)],
                         acc.at[pl.ds(s * _RPS + r0b, nb)], g0)
    for r0b, nb in _BLOCKS:
        pltpu.make_async_copy(rows0.at[pl.ds(0, nb)],
                              acc.at[pl.ds(s * _RPS + r0b, nb)], g0).wait()
    plsc.subcore_barrier()

    e0 = wid * _EPW
    for k, (rw, isk, idk, gk, sk) in enumerate(slots):
        off = e0 + k * _CH
        pltpu.sync_copy(src_hbm.at[pl.ds(off, _CH)], isk)
        pltpu.sync_copy(dst_hbm.at[pl.ds(off, _CH)], idk)
        pltpu.async_copy(h_hbm.at[isk], rw, gk)

    def _iter(i, _):
        for rw, isk, idk, gk, sk in slots:
            pltpu.make_async_copy(h_hbm.at[isk], rw, gk).wait()
            pltpu.async_copy(rw, acc.at[idk], sk, add=True)
        for k, (rw, isk, idk, gk, sk) in enumerate(slots):
            jn = _NSLOT * i + k + _NSLOT

            @pl.when(jn < _NCHUNK)
            def _(rw=rw, isk=isk, idk=idk, gk=gk, sk=sk, jn=jn):
                pltpu.make_async_copy(rw, acc.at[idk], sk).wait()
                off = e0 + jn * _CH
                pltpu.sync_copy(src_hbm.at[pl.ds(off, _CH)], isk)
                pltpu.sync_copy(dst_hbm.at[pl.ds(off, _CH)], idk)
                pltpu.async_copy(h_hbm.at[isk], rw, gk)
        return 0
    lax.fori_loop(0, _NCHUNK // _NSLOT, _iter, 0)
    for rw, isk, idk, gk, sk in slots:
        pltpu.make_async_copy(rw, acc.at[idk], sk).wait()

    ot = e0 + _NCHUNK * _CH
    pltpu.sync_copy(src_hbm.at[pl.ds(ot, _TAIL)], ist)
    pltpu.sync_copy(dst_hbm.at[pl.ds(ot, _TAIL)], idt)
    pltpu.async_copy(h_hbm.at[ist], rows0.at[pl.ds(0, _TAIL)], g0).wait()
    pltpu.sync_copy(rows0.at[pl.ds(0, _TAIL)], acc.at[idt], add=True)
    plsc.subcore_barrier()

    def _wb_in(idx):
        r0b, nb = _BLOCKS[idx]
        rw = slots[idx % _NSLOT][0]
        return (acc.at[pl.ds(s * _RPS + r0b, nb)], rw.at[pl.ds(0, nb)],
                slots[idx % _NSLOT][3])
    def _wb_out(idx):
        r0b, nb = _BLOCKS[idx]
        rw = slots[idx % _NSLOT][0]
        return (rw.at[pl.ds(0, nb)], out_hbm.at[c, pl.ds(s * _RPS + r0b, nb)],
                slots[idx % _NSLOT][4])
    for idx in range(len(_BLOCKS)):
        if idx >= _NSLOT:
            pltpu.make_async_copy(*_wb_out(idx - _NSLOT)).wait()
        pltpu.async_copy(*_wb_in(idx))
        pltpu.make_async_copy(*_wb_in(idx)).wait()
        pltpu.async_copy(*_wb_out(idx))
    for idx in range(max(0, len(_BLOCKS) - _NSLOT), len(_BLOCKS)):
        pltpu.make_async_copy(*_wb_out(idx)).wait()


def _segsum(h, src, dst):
    mesh = plsc.VectorSubcoreMesh(core_axis_name="c", subcore_axis_name="s")
    f = pl.kernel(
        _segsum_body,
        mesh=mesh,
        out_type=jax.ShapeDtypeStruct((_NC, _ROWS, _D), jnp.float32),
        scratch_types=[
            pltpu.VMEM_SHARED((_ROWS, _D), jnp.float32),
        ] + [
            t for _k in range(_NSLOT) for t in (
                pltpu.VMEM((_CH, _D), jnp.float32),
                pltpu.VMEM((_CH,), jnp.int32),
                pltpu.VMEM((_CH,), jnp.int32),
                pltpu.SemaphoreType.DMA,
                pltpu.SemaphoreType.DMA,
            )
        ] + [
            pltpu.VMEM((_TAIL,), jnp.int32),
            pltpu.VMEM((_TAIL,), jnp.int32),
        ],
    )
    return f(h, src, dst)


def _matmul_t(x, w):
    return lax.dot_general(x, w, (((1,), (1,)), ((), ())),
                           preferred_element_type=jnp.float32)


def _lin_body(x_ref, w_ref, b_ref, o_ref):
    o_ref[...] = _matmul_t(x_ref[...], w_ref[...]) + b_ref[...]


def _lin(x, p):
    return pl.pallas_call(
        _lin_body,
        out_shape=jax.ShapeDtypeStruct((_N, _D), jnp.float32),
    )(x, p["W"], p["b"].reshape(1, -1))


def _mlp_core(x_ref, a_ref, eps_ref, w1_ref, b1_ref, g_ref, be_ref, w2_ref, b2_ref):
    agg = a_ref[0, 0:_N, :] + a_ref[1, 0:_N, :]
    h = (1.0 + eps_ref[0]) * x_ref[...] + agg
    t = _matmul_t(h, w1_ref[...]) + b1_ref[...]
    mean = jnp.mean(t, axis=0, keepdims=True)
    var = jnp.mean((t - mean) ** 2, axis=0, keepdims=True)
    t = (t - mean) * lax.rsqrt(var + 1e-5) * g_ref[...] + be_ref[...]
    t = jnp.maximum(t, 0.0)
    t = _matmul_t(t, w2_ref[...]) + b2_ref[...]
    return jnp.maximum(t, 0.0)


def _gin_mlp_body(x_ref, a_ref, eps_ref, w1_ref, b1_ref, g_ref, be_ref,
                  w2_ref, b2_ref, o_ref):
    o_ref[...] = _mlp_core(x_ref, a_ref, eps_ref, w1_ref, b1_ref, g_ref,
                           be_ref, w2_ref, b2_ref)


def _gin_mlp_final_body(x_ref, a_ref, eps_ref, w1_ref, b1_ref, g_ref, be_ref,
                        w2_ref, b2_ref, wf_ref, bf_ref, o_ref):
    t = _mlp_core(x_ref, a_ref, eps_ref, w1_ref, b1_ref, g_ref,
                  be_ref, w2_ref, b2_ref)
    o_ref[...] = _matmul_t(t, wf_ref[...]) + bf_ref[...]


def _mlp_args(x, agg, p):
    return (x, agg, p["eps"].reshape(1),
            p["W1"], p["b1"].reshape(1, -1),
            p["gamma"].reshape(1, -1), p["beta"].reshape(1, -1),
            p["W2"], p["b2"].reshape(1, -1))


_SMEM1 = pl.BlockSpec(memory_space=pltpu.SMEM)


def _gin_mlp(x, agg, p):
    specs = [None, None, _SMEM1] + [None] * 6
    specs = [s if s is not None else pl.BlockSpec() for s in specs]
    return pl.pallas_call(
        _gin_mlp_body,
        in_specs=specs,
        out_shape=jax.ShapeDtypeStruct((_N, _D), jnp.float32),
    )(*_mlp_args(x, agg, p))


def _gin_mlp_final(x, agg, p, pf):
    specs = [None, None, _SMEM1] + [None] * 8
    specs = [s if s is not None else pl.BlockSpec() for s in specs]
    return pl.pallas_call(
        _gin_mlp_final_body,
        in_specs=specs,
        out_shape=jax.ShapeDtypeStruct((_N, pf["W"].shape[0]), jnp.float32),
    )(*_mlp_args(x, agg, p), pf["W"], pf["b"].reshape(1, -1))


def kernel(x_author, x_paper, params, ei_writes, ei_written):
    p = params
    src_w, dst_w = ei_writes[0], ei_writes[1]
    src_n, dst_n = ei_written[0], ei_written[1]

    h_a = _lin(x_author, p["lin_author"])
    h_p = _lin(x_paper, p["lin_paper"])

    l1, l2 = p["layers"][0], p["layers"][1]
    agg_p = _segsum(h_a, src_w, dst_w)
    agg_a = _segsum(h_p, src_n, dst_n)
    h_p1 = _gin_mlp(h_p, agg_p, l1["writes"])
    h_a1 = _gin_mlp(h_a, agg_a, l1["written"])

    agg_a2 = _segsum(h_p1, src_n, dst_n)
    return _gin_mlp_final(h_a1, agg_a2, l2["written"], p["final"])
